# baseline (speedup 1.0000x reference)
import functools

import jax
import jax.numpy as jnp
from jax import lax
from jax.experimental import pallas as pl
from jax.experimental.pallas import tpu as pltpu
from jax.experimental.pallas import tpu_sc as plsc

B = 4096
C = 8
E = 128
H = 256
K = 20

NC = 2
NS = 16
NW = NC * NS
LANES = 16

_MESH = plsc.VectorSubcoreMesh(core_axis_name="c", subcore_axis_name="s")
_SC_TILED = pltpu.CompilerParams(use_tc_tiling_on_sc=True,
                                 needs_layout_passes=False)
_SC_LINEAR = pltpu.CompilerParams(use_tc_tiling_on_sc=False,
                                  needs_layout_passes=False)


def _wid():
    return lax.axis_index("s") * NC + lax.axis_index("c")


_CTX_ROWS = B * C
_ROWS_PER_W = _CTX_ROWS // NW
_CH = 128
_NCH = _ROWS_PER_W // _CH


@functools.partial(
    pl.kernel,
    mesh=_MESH,
    out_type=jax.ShapeDtypeStruct((_CTX_ROWS, E), jnp.float32),
    compiler_params=_SC_LINEAR,
    scratch_types=[
        pltpu.VMEM((_NCH, _CH), jnp.int32),
        pltpu.VMEM((2, _CH, E), jnp.float32),
        pltpu.SemaphoreType.DMA,
    ],
)
def _gather_ctx(idx_hbm, table_hbm, out_hbm, idx_v, rows_v, sem):
    wid = _wid()
    pltpu.sync_copy(idx_hbm.at[pl.ds(wid * _NCH, _NCH)], idx_v)
    cps = [None, None]
    for i in range(_NCH + 2):
        if i >= 2:
            cps[i % 2].wait()
            pltpu.sync_copy(
                rows_v.at[i % 2],
                out_hbm.at[pl.ds(wid * _ROWS_PER_W + (i - 2) * _CH, _CH)])
        if i < _NCH:
            cps[i % 2] = pltpu.async_copy(
                table_hbm.at[idx_v.at[i]], rows_v.at[i % 2], sem)


_BM = 512


def _mlp_body(x_ref, w_ref, b_ref, o_ref):
    x = jnp.concatenate([x_ref[c] for c in range(C)], axis=1)
    o_ref[...] = jnp.tanh(
        jnp.dot(x, w_ref[...], preferred_element_type=jnp.float32)
        + b_ref[...])


_mlp = pl.pallas_call(
    _mlp_body,
    grid=(B // _BM,),
    in_specs=[
        pl.BlockSpec((C, _BM, E), lambda i: (0, i, 0)),
        pl.BlockSpec((C * E, H), lambda i: (0, 0)),
        pl.BlockSpec((1, H), lambda i: (0, 0)),
    ],
    out_specs=pl.BlockSpec((_BM, H), lambda i: (i, 0)),
    out_shape=jax.ShapeDtypeStruct((B, H), jnp.float32),
)


_SPW = B // NW
_SG = 16
_NG = _SPW // _SG
_KT = K + 1
_CS = 4
_CPW = _SPW // _CS
_CR = _CS * _KT
_PKT = 88
_HCH = H // LANES


def _softplus(x):
    u = jnp.exp(-jnp.abs(x))
    t = u / (2.0 + u)
    t2 = t * t
    return jnp.maximum(x, 0.0) + 2.0 * t * (1.0 + t2 * (1.0 / 3.0 + t2 * 0.2))


@functools.partial(
    pl.kernel,
    mesh=_MESH,
    out_type=jax.ShapeDtypeStruct((NW * LANES,), jnp.float32),
    compiler_params=_SC_TILED,
    scratch_types=[
        pltpu.VMEM((_SPW // 2, H), jnp.float32),
        pltpu.VMEM((_SPW * _KT + 16,), jnp.int32),
        pltpu.VMEM((_CPW * _PKT + 16,), jnp.int32),
        pltpu.VMEM((4, _PKT, H), jnp.float32),
        pltpu.VMEM((LANES,), jnp.float32),
        pltpu.SemaphoreType.DMA,
        pltpu.SemaphoreType.DMA,
        pltpu.SemaphoreType.DMA,
        pltpu.SemaphoreType.DMA,
    ],
)
def _dots(hid_hbm, table_hbm, tidx_hbm, part_hbm,
          hid_v, idx_raw, idx_pad, rows_v, part_v, s0, s1, s2, s3):
    wid = _wid()
    lanes = lax.iota(jnp.int32, LANES)
    sems = (s0, s1, s2, s3)

    def chunk_copy(j, slot):
        return pltpu.make_async_copy(
            table_hbm.at[idx_pad.at[pl.ds(j * _PKT, _PKT)]],
            rows_v.at[slot], sems[slot])

    pltpu.sync_copy(tidx_hbm.at[pl.ds(wid * _SPW * _KT, _SPW * _KT)],
                    idx_raw.at[pl.ds(0, _SPW * _KT)])
    for j in range(_CPW):
        for c in range(6):
            v = plsc.load_gather(idx_raw, [lanes + (j * _CR + c * LANES)])
            if c == 5:
                v = jnp.where((lanes >= 4) & (lanes < 8), 0, v)
            idx_pad[pl.ds(j * _PKT + c * LANES, LANES)] = v
    for slot in range(4):
        chunk_copy(slot, slot).start()
    pltpu.sync_copy(hid_hbm.at[pl.ds(wid * _SPW, _SPW // 2)], hid_v)

    def gbody(g, loss_acc):
        @pl.when(g == _NG // 2)
        def _():
            pltpu.sync_copy(
                hid_hbm.at[pl.ds(wid * _SPW + _SPW // 2, _SPW // 2)], hid_v)

        hbase = lax.rem(g, _NG // 2) * _SG
        res = tuple(jnp.zeros((LANES,), jnp.float32) for _ in range(_KT))
        for j2 in range(4):
            j = g * 4 + j2
            chunk_copy(j, j2).wait()

            def body(sl, res, j2=j2):
                s = j2 * _CS + sl
                sel = lanes == s
                hrow = hbase + s
                hc = [hid_v[hrow, pl.ds(c * LANES, LANES)]
                      for c in range(_HCH)]

                def dot_row(r):
                    acc = hc[0] * rows_v[j2, r, pl.ds(0, LANES)]
                    for c in range(1, _HCH):
                        acc += hc[c] * rows_v[j2, r, pl.ds(c * LANES, LANES)]
                    return jnp.sum(acc)

                return tuple(
                    jnp.where(sel, dot_row(sl * _KT + k), res[k])
                    for k in range(_KT))

            res = lax.fori_loop(0, _CS, body, res)

            @pl.when(j < _CPW - 4)
            def _(j=j, j2=j2):
                chunk_copy(j + 4, j2).start()

        loss_acc += _softplus(-res[0])
        for k in range(K):
            loss_acc += _softplus(res[k + 1])
        return loss_acc

    part_v[...] = lax.fori_loop(0, _NG, gbody, jnp.zeros((LANES,), jnp.float32))
    pltpu.sync_copy(part_v, part_hbm.at[pl.ds(wid * LANES, LANES)])


def kernel(in_embed, out_embed, W1, b1, center, context, neg_context):
    ctx_idx = context.T.reshape(_CTX_ROWS // _CH, _CH).astype(jnp.int32)
    ctx_rows = _gather_ctx(ctx_idx, in_embed)
    hidden = _mlp(ctx_rows.reshape(C, B, E), W1.T, b1.reshape(1, H))
    tidx = jnp.concatenate(
        [center.reshape(B, 1), neg_context], axis=1).reshape(-1)
    partials = _dots(hidden, out_embed, tidx.astype(jnp.int32))
    return jnp.sum(partials) * (1.0 / B)


# speedup vs baseline: 2.2364x; 2.2364x over previous
import functools

import jax
import jax.numpy as jnp
from jax import lax
from jax.experimental import pallas as pl
from jax.experimental.pallas import tpu as pltpu
from jax.experimental.pallas import tpu_sc as plsc

B = 4096
C = 8
E = 128
H = 256
K = 20

NC = 2
NS = 16
NW = NC * NS
LANES = 16

_MESH = plsc.VectorSubcoreMesh(core_axis_name="c", subcore_axis_name="s")
_SC_TILED = pltpu.CompilerParams(use_tc_tiling_on_sc=True,
                                 needs_layout_passes=False)
_SC_LINEAR = pltpu.CompilerParams(use_tc_tiling_on_sc=False,
                                  needs_layout_passes=False)


def _wid():
    return lax.axis_index("s") * NC + lax.axis_index("c")


_CTX_ROWS = B * C
_ROWS_PER_W = _CTX_ROWS // NW
_CH = 128
_NCH = _ROWS_PER_W // _CH


@functools.partial(
    pl.kernel,
    mesh=_MESH,
    out_type=jax.ShapeDtypeStruct((_CTX_ROWS, E), jnp.float32),
    compiler_params=_SC_LINEAR,
    scratch_types=[
        pltpu.VMEM((_NCH, _CH), jnp.int32),
        pltpu.VMEM((2, _CH, E), jnp.float32),
        pltpu.SemaphoreType.DMA,
    ],
)
def _gather_ctx(idx_hbm, table_hbm, out_hbm, idx_v, rows_v, sem):
    wid = _wid()
    pltpu.sync_copy(idx_hbm.at[pl.ds(wid * _NCH, _NCH)], idx_v)
    cps = [None, None]
    for i in range(_NCH + 2):
        if i >= 2:
            cps[i % 2].wait()
            pltpu.sync_copy(
                rows_v.at[i % 2],
                out_hbm.at[pl.ds(wid * _ROWS_PER_W + (i - 2) * _CH, _CH)])
        if i < _NCH:
            cps[i % 2] = pltpu.async_copy(
                table_hbm.at[idx_v.at[i]], rows_v.at[i % 2], sem)


_BM = 512


def _mlp_body(x_ref, w_ref, b_ref, o_ref):
    x = jnp.concatenate([x_ref[c] for c in range(C)], axis=1)
    o_ref[...] = jnp.tanh(
        jnp.dot(x, w_ref[...], preferred_element_type=jnp.float32)
        + b_ref[...])


_mlp = pl.pallas_call(
    _mlp_body,
    grid=(B // _BM,),
    in_specs=[
        pl.BlockSpec((C, _BM, E), lambda i: (0, i, 0)),
        pl.BlockSpec((C * E, H), lambda i: (0, 0)),
        pl.BlockSpec((1, H), lambda i: (0, 0)),
    ],
    out_specs=pl.BlockSpec((_BM, H), lambda i: (i, 0)),
    out_shape=jax.ShapeDtypeStruct((B, H), jnp.float32),
)


_SPW = B // NW
_SG = 16
_NG = _SPW // _SG
_KT = K + 1
_HG = 8 * _KT
_HCH = H // LANES


def _softplus(x):
    u = jnp.exp(-jnp.abs(x))
    t = u / (2.0 + u)
    t2 = t * t
    return jnp.maximum(x, 0.0) + 2.0 * t * (1.0 + t2 * (1.0 / 3.0 + t2 * 0.2))


@functools.partial(
    pl.kernel,
    mesh=_MESH,
    out_type=jax.ShapeDtypeStruct((NW * LANES,), jnp.float32),
    compiler_params=_SC_TILED,
    scratch_types=[
        pltpu.VMEM((_SPW, H), jnp.float32),
        pltpu.VMEM((_SPW * _KT,), jnp.int32),
        pltpu.VMEM((2, _HG, H), jnp.float32),
        pltpu.VMEM((LANES,), jnp.float32),
        pltpu.SemaphoreType.DMA,
        pltpu.SemaphoreType.DMA,
    ],
)
def _dots(hid_hbm, table_hbm, tidx_hbm, part_hbm,
          hid_v, tidx_v, rows_v, part_v, sem0, sem1):
    wid = _wid()
    lanes = lax.iota(jnp.int32, LANES)
    sems = (sem0, sem1)

    def slot_copies(g, h):
        base = g * (_SG * _KT) + h * _HG
        return (
            pltpu.make_async_copy(
                table_hbm.at[tidx_v.at[pl.ds(base, 128)]],
                rows_v.at[h, pl.ds(0, 128)], sems[h]),
            pltpu.make_async_copy(
                table_hbm.at[tidx_v.at[pl.ds(base + 128, _HG - 128)]],
                rows_v.at[h, pl.ds(128, _HG - 128)], sems[h]),
        )

    def fire(g, h):
        for cp in slot_copies(g, h):
            cp.start()

    pltpu.sync_copy(tidx_hbm.at[pl.ds(wid * _SPW * _KT, _SPW * _KT)], tidx_v)
    fire(0, 0)
    fire(0, 1)
    pltpu.sync_copy(hid_hbm.at[pl.ds(wid * _SPW, _SPW)], hid_v)

    def gbody(g, loss_acc):
        res = tuple(jnp.zeros((LANES,), jnp.float32) for _ in range(_KT))
        for h in (0, 1):
            for cp in slot_copies(g, h):
                cp.wait()

            def body(sl, res, h=h):
                s = h * 8 + sl
                sel = lanes == s
                hrow = g * _SG + s
                hc = [hid_v[hrow, pl.ds(c * LANES, LANES)]
                      for c in range(_HCH)]

                def dot_row(r):
                    acc = hc[0] * rows_v[h, r, pl.ds(0, LANES)]
                    for c in range(1, _HCH):
                        acc += hc[c] * rows_v[h, r, pl.ds(c * LANES, LANES)]
                    return jnp.sum(acc)

                return tuple(
                    jnp.where(sel, dot_row(sl * _KT + k), res[k])
                    for k in range(_KT))

            res = lax.fori_loop(0, 8, body, res)

            @pl.when(g < _NG - 1)
            def _(h=h):
                fire(g + 1, h)

        loss_acc += _softplus(-res[0])
        for k in range(K):
            loss_acc += _softplus(res[k + 1])
        return loss_acc

    part_v[...] = lax.fori_loop(0, _NG, gbody, jnp.zeros((LANES,), jnp.float32))
    pltpu.sync_copy(part_v, part_hbm.at[pl.ds(wid * LANES, LANES)])


def kernel(in_embed, out_embed, W1, b1, center, context, neg_context):
    ctx_idx = context.T.reshape(_CTX_ROWS // _CH, _CH).astype(jnp.int32)
    ctx_rows = _gather_ctx(ctx_idx, in_embed)
    hidden = _mlp(ctx_rows.reshape(C, B, E), W1.T, b1.reshape(1, H))
    tidx = jnp.concatenate(
        [center.reshape(B, 1), neg_context], axis=1).reshape(-1)
    partials = _dots(hidden, out_embed, tidx.astype(jnp.int32))
    return jnp.sum(partials) * (1.0 / B)
